# Initial kernel scaffold; baseline (speedup 1.0000x reference)
import jax
import jax.numpy as jnp
from jax.experimental import pallas as pl


def kernel(x, edge_index, W1, b1, W2, b2, Wl, bl):
    raise NotImplementedError("write your pallas kernel here")


# speedup vs baseline: 14.9278x; 14.9278x over previous
import functools

import jax
import jax.numpy as jnp
from jax import lax
from jax.experimental import pallas as pl
from jax.experimental.pallas import tpu as pltpu
from jax.experimental.pallas import tpu_sc as plsc

N = 10000
D = 128
H = 64
E = 320000

NC = 2
NS = 16
NW = NC * NS
C = 128
CH = 79
EP = NW * CH * C
PAD = EP - E
NCHUNK = NW * CH
RPT = 640
NRP = NS * RPT
DUMP = N

_mesh = plsc.VectorSubcoreMesh(
    core_axis_name="c", subcore_axis_name="s", num_cores=NC, num_subcores=NS
)

_sc_params = pltpu.CompilerParams(use_tc_tiling_on_sc=False)


@functools.partial(
    pl.kernel,
    out_type=jax.ShapeDtypeStruct((NC, NRP), jnp.float32),
    mesh=_mesh,
    scratch_types=[
        pltpu.VMEM((C,), jnp.int32),
        pltpu.VMEM((C,), jnp.float32),
        pltpu.VMEM((RPT,), jnp.float32),
        pltpu.VMEM_SHARED((NRP,), jnp.float32),
    ],
    compiler_params=_sc_params,
)
def _deg_kernel(dst2d, out, idx_v, ones_v, zb_v, acc):
    c = lax.axis_index("c")
    s = lax.axis_index("s")
    wid = c * NS + s
    one = jnp.ones((16,), jnp.float32)
    zero = jnp.zeros((16,), jnp.float32)
    for q in range(C // 16):
        ones_v[pl.ds(q * 16, 16)] = one
    for q in range(RPT // 16):
        zb_v[pl.ds(q * 16, 16)] = zero
    pltpu.sync_copy(zb_v, acc.at[pl.ds(s * RPT, RPT)])
    plsc.subcore_barrier()

    def body(j, carry):
        row = wid * CH + j
        pltpu.sync_copy(dst2d.at[row], idx_v)
        pltpu.sync_copy(ones_v, acc.at[idx_v], add=True)
        return carry

    lax.fori_loop(0, CH, body, 0)
    plsc.subcore_barrier()
    pltpu.sync_copy(acc.at[pl.ds(s * RPT, RPT)], out.at[c, pl.ds(s * RPT, RPT)])


@functools.partial(
    pl.kernel,
    out_type=jax.ShapeDtypeStruct((NC, NRP, H), jnp.float32),
    mesh=_mesh,
    scratch_types=[
        pltpu.VMEM((C,), jnp.int32),
        pltpu.VMEM((C,), jnp.int32),
        pltpu.VMEM((C, H), jnp.float32),
        pltpu.VMEM((64, H), jnp.float32),
        pltpu.VMEM_SHARED((NRP, H), jnp.float32),
        pltpu.SemaphoreType.DMA,
    ],
    compiler_params=_sc_params,
)
def _agg_kernel(hp, src2d, dst2d, out, src_v, dst_v, rows_v, zb_v, acc, sem):
    c = lax.axis_index("c")
    s = lax.axis_index("s")
    wid = c * NS + s
    zero = jnp.zeros((16,), jnp.float32)
    for r in range(64):
        for q in range(H // 16):
            zb_v[r, pl.ds(q * 16, 16)] = zero
    for k in range(RPT // 64):
        pltpu.sync_copy(zb_v, acc.at[pl.ds(s * RPT + k * 64, 64)])
    plsc.subcore_barrier()

    def body(j, carry):
        row = wid * CH + j
        pltpu.sync_copy(src2d.at[row], src_v)
        pltpu.sync_copy(dst2d.at[row], dst_v)
        pltpu.async_copy(hp.at[src_v], rows_v, sem).wait()
        pltpu.sync_copy(rows_v, acc.at[dst_v], add=True)
        return carry

    lax.fori_loop(0, CH, body, 0)
    plsc.subcore_barrier()
    pltpu.sync_copy(acc.at[pl.ds(s * RPT, RPT)], out.at[c, pl.ds(s * RPT, RPT)])


def _pre_body(degp_ref, x_ref, w1_ref, hp_ref, dinv_ref):
    deg = degp_ref[0, :N, :] + degp_ref[1, :N, :] + 1.0
    dinv = lax.rsqrt(deg)
    h = jnp.dot(x_ref[...], w1_ref[...], preferred_element_type=jnp.float32)
    hp_ref[...] = h * dinv
    dinv_ref[...] = dinv


_pre_call = pl.pallas_call(
    _pre_body,
    out_shape=(
        jax.ShapeDtypeStruct((N, H), jnp.float32),
        jax.ShapeDtypeStruct((N, 1), jnp.float32),
    ),
)


def _mid_body(agg_ref, hp1_ref, dinv_ref, b1_ref, w2_ref, hp2_ref):
    a = agg_ref[0, :N, :] + agg_ref[1, :N, :]
    dinv = dinv_ref[...]
    z1 = jnp.maximum(dinv * (a + hp1_ref[...]) + b1_ref[...], 0.0)
    hp2_ref[...] = jnp.dot(z1, w2_ref[...], preferred_element_type=jnp.float32) * dinv


_mid_call = pl.pallas_call(
    _mid_body,
    out_shape=jax.ShapeDtypeStruct((N, H), jnp.float32),
)


def _post_body(agg_ref, hp2_ref, dinv_ref, b2_ref, wl_ref, bl_ref, y_ref):
    a = agg_ref[0, :N, :] + agg_ref[1, :N, :]
    dinv = dinv_ref[...]
    z2 = jnp.maximum(dinv * (a + hp2_ref[...]) + b2_ref[...], 0.0)
    y_ref[...] = jnp.dot(z2, wl_ref[...], preferred_element_type=jnp.float32) + bl_ref[...]


_post_call = pl.pallas_call(
    _post_body,
    out_shape=jax.ShapeDtypeStruct((N, 1), jnp.float32),
)


def kernel(x, edge_index, W1, b1, W2, b2, Wl, bl):
    src = edge_index[0]
    dst = edge_index[1]
    src_p = jnp.concatenate([src, jnp.zeros((PAD,), jnp.int32)]).reshape(NCHUNK, C)
    dst_p = jnp.concatenate([dst, jnp.full((PAD,), DUMP, jnp.int32)]).reshape(NCHUNK, C)

    degp = _deg_kernel(dst_p).reshape(NC, NRP, 1)
    hp1, dinv = _pre_call(degp, x, W1)
    agg1 = _agg_kernel(hp1, src_p, dst_p)
    hp2 = _mid_call(agg1, hp1, dinv, b1.reshape(1, H), W2)
    agg2 = _agg_kernel(hp2, src_p, dst_p)
    y = _post_call(agg2, hp2, dinv, b2.reshape(1, H), Wl, bl.reshape(1, 1))
    return y


# speedup vs baseline: 16.5734x; 1.1102x over previous
import functools

import jax
import jax.numpy as jnp
from jax import lax
from jax.experimental import pallas as pl
from jax.experimental.pallas import tpu as pltpu
from jax.experimental.pallas import tpu_sc as plsc

N = 10000
D = 128
H = 64
E = 320000

NC = 2
NS = 16
NW = NC * NS
C = 128
CH = 80
EP = NW * CH * C
PAD = EP - E
NCHUNK = NW * CH
RPT = 640
NRP = NS * RPT
DUMP = N

_mesh = plsc.VectorSubcoreMesh(
    core_axis_name="c", subcore_axis_name="s", num_cores=NC, num_subcores=NS
)

_sc_params = pltpu.CompilerParams(use_tc_tiling_on_sc=False)


@functools.partial(
    pl.kernel,
    out_type=jax.ShapeDtypeStruct((NC, NRP), jnp.float32),
    mesh=_mesh,
    scratch_types=[
        pltpu.VMEM((CH, C), jnp.int32),
        pltpu.VMEM((C,), jnp.float32),
        pltpu.VMEM((RPT,), jnp.float32),
        pltpu.VMEM_SHARED((NRP,), jnp.float32),
    ],
    compiler_params=_sc_params,
)
def _deg_kernel(dst2d, out, idxs_v, ones_v, zb_v, acc):
    c = lax.axis_index("c")
    s = lax.axis_index("s")
    wid = c * NS + s
    one = jnp.ones((16,), jnp.float32)
    zero = jnp.zeros((16,), jnp.float32)
    for q in range(C // 16):
        ones_v[pl.ds(q * 16, 16)] = one
    for q in range(RPT // 16):
        zb_v[pl.ds(q * 16, 16)] = zero
    pltpu.sync_copy(dst2d.at[pl.ds(wid * CH, CH)], idxs_v)
    pltpu.sync_copy(zb_v, acc.at[pl.ds(s * RPT, RPT)])
    plsc.subcore_barrier()

    def body(j, carry):
        pltpu.sync_copy(ones_v, acc.at[idxs_v.at[j]], add=True)
        return carry

    lax.fori_loop(0, CH, body, 0)
    plsc.subcore_barrier()
    pltpu.sync_copy(acc.at[pl.ds(s * RPT, RPT)], out.at[c, pl.ds(s * RPT, RPT)])


@functools.partial(
    pl.kernel,
    out_type=jax.ShapeDtypeStruct((NC, NRP, H), jnp.float32),
    mesh=_mesh,
    scratch_types=[
        pltpu.VMEM((CH, C), jnp.int32),
        pltpu.VMEM((CH, C), jnp.int32),
        pltpu.VMEM((C, H), jnp.float32),
        pltpu.VMEM((C, H), jnp.float32),
        pltpu.VMEM((64, H), jnp.float32),
        pltpu.VMEM_SHARED((NRP, H), jnp.float32),
        pltpu.SemaphoreType.DMA,
        pltpu.SemaphoreType.DMA,
    ],
    compiler_params=_sc_params,
)
def _agg_kernel(hp, src2d, dst2d, out, srcs_v, dsts_v, rows0, rows1, zb_v, acc,
                sem0, sem1):
    c = lax.axis_index("c")
    s = lax.axis_index("s")
    wid = c * NS + s
    zero = jnp.zeros((16,), jnp.float32)
    for r in range(64):
        for q in range(H // 16):
            zb_v[r, pl.ds(q * 16, 16)] = zero
    pltpu.sync_copy(src2d.at[pl.ds(wid * CH, CH)], srcs_v)
    pltpu.sync_copy(dst2d.at[pl.ds(wid * CH, CH)], dsts_v)
    for k in range(RPT // 64):
        pltpu.sync_copy(zb_v, acc.at[pl.ds(s * RPT + k * 64, 64)])
    plsc.subcore_barrier()

    pltpu.async_copy(hp.at[srcs_v.at[0]], rows0, sem0)
    npair = CH // 2

    def body(i, carry):
        j = 2 * i
        pltpu.async_copy(hp.at[srcs_v.at[j + 1]], rows1, sem1)
        pltpu.make_async_copy(hp.at[srcs_v.at[j]], rows0, sem0).wait()
        pltpu.sync_copy(rows0, acc.at[dsts_v.at[j]], add=True)

        @pl.when(i < npair - 1)
        def _():
            pltpu.async_copy(hp.at[srcs_v.at[j + 2]], rows0, sem0)

        pltpu.make_async_copy(hp.at[srcs_v.at[j + 1]], rows1, sem1).wait()
        pltpu.sync_copy(rows1, acc.at[dsts_v.at[j + 1]], add=True)
        return carry

    lax.fori_loop(0, npair, body, 0)
    plsc.subcore_barrier()
    pltpu.sync_copy(acc.at[pl.ds(s * RPT, RPT)], out.at[c, pl.ds(s * RPT, RPT)])


def _pre_body(degp_ref, x_ref, w1_ref, hp_ref, dinv_ref):
    deg = degp_ref[0, :N, :] + degp_ref[1, :N, :] + 1.0
    dinv = lax.rsqrt(deg)
    h = jnp.dot(x_ref[...], w1_ref[...], preferred_element_type=jnp.float32)
    hp_ref[...] = h * dinv
    dinv_ref[...] = dinv


_pre_call = pl.pallas_call(
    _pre_body,
    out_shape=(
        jax.ShapeDtypeStruct((N, H), jnp.float32),
        jax.ShapeDtypeStruct((N, 1), jnp.float32),
    ),
)


def _mid_body(agg_ref, hp1_ref, dinv_ref, b1_ref, w2_ref, hp2_ref):
    a = agg_ref[0, :N, :] + agg_ref[1, :N, :]
    dinv = dinv_ref[...]
    z1 = jnp.maximum(dinv * (a + hp1_ref[...]) + b1_ref[...], 0.0)
    hp2_ref[...] = jnp.dot(z1, w2_ref[...], preferred_element_type=jnp.float32) * dinv


_mid_call = pl.pallas_call(
    _mid_body,
    out_shape=jax.ShapeDtypeStruct((N, H), jnp.float32),
)


def _post_body(agg_ref, hp2_ref, dinv_ref, b2_ref, wl_ref, bl_ref, y_ref):
    a = agg_ref[0, :N, :] + agg_ref[1, :N, :]
    dinv = dinv_ref[...]
    z2 = jnp.maximum(dinv * (a + hp2_ref[...]) + b2_ref[...], 0.0)
    y_ref[...] = jnp.dot(z2, wl_ref[...], preferred_element_type=jnp.float32) + bl_ref[...]


_post_call = pl.pallas_call(
    _post_body,
    out_shape=jax.ShapeDtypeStruct((N, 1), jnp.float32),
)


def kernel(x, edge_index, W1, b1, W2, b2, Wl, bl):
    src = edge_index[0]
    dst = edge_index[1]
    src_p = jnp.concatenate([src, jnp.zeros((PAD,), jnp.int32)]).reshape(NCHUNK, C)
    dst_p = jnp.concatenate([dst, jnp.full((PAD,), DUMP, jnp.int32)]).reshape(NCHUNK, C)

    degp = _deg_kernel(dst_p).reshape(NC, NRP, 1)
    hp1, dinv = _pre_call(degp, x, W1)
    agg1 = _agg_kernel(hp1, src_p, dst_p)
    hp2 = _mid_call(agg1, hp1, dinv, b1.reshape(1, H), W2)
    agg2 = _agg_kernel(hp2, src_p, dst_p)
    y = _post_call(agg2, hp2, dinv, b2.reshape(1, H), Wl, bl.reshape(1, 1))
    return y


# speedup vs baseline: 39.4734x; 2.3817x over previous
import functools

import jax
import jax.numpy as jnp
from jax import lax
from jax.experimental import pallas as pl
from jax.experimental.pallas import tpu as pltpu
from jax.experimental.pallas import tpu_sc as plsc

N = 10000
D = 128
H = 64
E = 320000

NC = 2
NS = 16
NW = NC * NS
C = 128
CH = 80
EP = NW * CH * C
PAD = EP - E
NCHUNK = NW * CH
RPT = 640
NRP = NS * RPT
DUMP = N

_mesh = plsc.VectorSubcoreMesh(
    core_axis_name="c", subcore_axis_name="s", num_cores=NC, num_subcores=NS
)

_sc_params = pltpu.CompilerParams(use_tc_tiling_on_sc=False)


@functools.partial(
    pl.kernel,
    out_type=jax.ShapeDtypeStruct((NC, NRP), jnp.float32),
    mesh=_mesh,
    scratch_types=[
        pltpu.VMEM((CH, C), jnp.int32),
        pltpu.VMEM((C,), jnp.float32),
        pltpu.VMEM((RPT,), jnp.float32),
        pltpu.VMEM_SHARED((NRP,), jnp.float32),
    ],
    compiler_params=_sc_params,
)
def _deg_kernel(dst2d, out, idxs_v, ones_v, zb_v, acc):
    c = lax.axis_index("c")
    s = lax.axis_index("s")
    wid = c * NS + s
    one = jnp.ones((16,), jnp.float32)
    zero = jnp.zeros((16,), jnp.float32)
    for q in range(C // 16):
        ones_v[pl.ds(q * 16, 16)] = one
    for q in range(RPT // 16):
        zb_v[pl.ds(q * 16, 16)] = zero
    pltpu.sync_copy(dst2d.at[pl.ds(wid * CH, CH)], idxs_v)
    pltpu.sync_copy(zb_v, acc.at[pl.ds(s * RPT, RPT)])
    plsc.subcore_barrier()

    def body(j, carry):
        pltpu.sync_copy(ones_v, acc.at[idxs_v.at[j]], add=True)
        return carry

    lax.fori_loop(0, CH, body, 0)
    plsc.subcore_barrier()
    pltpu.sync_copy(acc.at[pl.ds(s * RPT, RPT)], out.at[c, pl.ds(s * RPT, RPT)])


@functools.partial(
    pl.kernel,
    out_type=jax.ShapeDtypeStruct((NC, NRP, H), jnp.float32),
    mesh=_mesh,
    scratch_types=[
        pltpu.VMEM((CH, C), jnp.int32),
        pltpu.VMEM((CH, C), jnp.int32),
        pltpu.VMEM((C, H), jnp.float32),
        pltpu.VMEM((C, H), jnp.float32),
        pltpu.VMEM((64, H), jnp.float32),
        pltpu.VMEM_SHARED((NRP, H), jnp.float32),
        pltpu.SemaphoreType.DMA,
        pltpu.SemaphoreType.DMA,
    ],
    compiler_params=_sc_params,
)
def _agg_kernel(hp, src2d, dst2d, out, srcs_v, dsts_v, rows0, rows1, zb_v, acc,
                sem0, sem1):
    c = lax.axis_index("c")
    s = lax.axis_index("s")
    wid = c * NS + s
    zero = jnp.zeros((16,), jnp.float32)
    for r in range(64):
        for q in range(H // 16):
            zb_v[r, pl.ds(q * 16, 16)] = zero
    pltpu.sync_copy(src2d.at[pl.ds(wid * CH, CH)], srcs_v)
    pltpu.sync_copy(dst2d.at[pl.ds(wid * CH, CH)], dsts_v)
    for k in range(RPT // 64):
        pltpu.sync_copy(zb_v, acc.at[pl.ds(s * RPT + k * 64, 64)])
    plsc.subcore_barrier()

    pltpu.async_copy(hp.at[srcs_v.at[0]], rows0, sem0)
    npair = CH // 2

    def body(i, carry):
        j = 2 * i
        pltpu.async_copy(hp.at[srcs_v.at[j + 1]], rows1, sem1)
        pltpu.make_async_copy(hp.at[srcs_v.at[j]], rows0, sem0).wait()
        pltpu.sync_copy(rows0, acc.at[dsts_v.at[j]], add=True)

        @pl.when(i < npair - 1)
        def _():
            pltpu.async_copy(hp.at[srcs_v.at[j + 2]], rows0, sem0)

        pltpu.make_async_copy(hp.at[srcs_v.at[j + 1]], rows1, sem1).wait()
        pltpu.sync_copy(rows1, acc.at[dsts_v.at[j + 1]], add=True)
        return carry

    lax.fori_loop(0, npair, body, 0)
    plsc.subcore_barrier()
    pltpu.sync_copy(acc.at[pl.ds(s * RPT, RPT)], out.at[c, pl.ds(s * RPT, RPT)])


def _pre_body(degp_ref, x_ref, w1_ref, hp_ref, dinv_ref):
    deg = degp_ref[0, :N, :] + degp_ref[1, :N, :] + 1.0
    dinv = lax.rsqrt(deg)
    h = jnp.dot(x_ref[...], w1_ref[...], preferred_element_type=jnp.float32)
    hp_ref[...] = h * dinv
    dinv_ref[...] = dinv


_pre_call = pl.pallas_call(
    _pre_body,
    out_shape=(
        jax.ShapeDtypeStruct((N, H), jnp.float32),
        jax.ShapeDtypeStruct((N, 1), jnp.float32),
    ),
)


def _mid_body(agg_ref, hp1_ref, dinv_ref, b1_ref, w2_ref, hp2_ref):
    a = agg_ref[0, :N, :] + agg_ref[1, :N, :]
    dinv = dinv_ref[...]
    z1 = jnp.maximum(dinv * (a + hp1_ref[...]) + b1_ref[...], 0.0)
    hp2_ref[...] = jnp.dot(z1, w2_ref[...], preferred_element_type=jnp.float32) * dinv


_mid_call = pl.pallas_call(
    _mid_body,
    out_shape=jax.ShapeDtypeStruct((N, H), jnp.float32),
)


def _post_body(agg_ref, hp2_ref, dinv_ref, b2_ref, wl_ref, bl_ref, y_ref):
    a = agg_ref[0, :N, :] + agg_ref[1, :N, :]
    dinv = dinv_ref[...]
    z2 = jnp.maximum(dinv * (a + hp2_ref[...]) + b2_ref[...], 0.0)
    y_ref[...] = jnp.dot(z2, wl_ref[...], preferred_element_type=jnp.float32) + bl_ref[...]


_post_call = pl.pallas_call(
    _post_body,
    out_shape=jax.ShapeDtypeStruct((N, 1), jnp.float32),
)


def kernel(x, edge_index, W1, b1, W2, b2, Wl, bl):
    src = edge_index[0]
    dst = edge_index[1]
    pad_i = jnp.arange(PAD, dtype=jnp.int32)
    src_pad = pad_i % N
    dst_pad = N + pad_i % (NRP - N)
    src_p = jnp.concatenate([src, src_pad]).reshape(NCHUNK, C)
    dst_p = jnp.concatenate([dst, dst_pad]).reshape(NCHUNK, C)

    degp = _deg_kernel(dst_p).reshape(NC, NRP, 1)
    hp1, dinv = _pre_call(degp, x, W1)
    agg1 = _agg_kernel(hp1, src_p, dst_p)
    hp2 = _mid_call(agg1, hp1, dinv, b1.reshape(1, H), W2)
    agg2 = _agg_kernel(hp2, src_p, dst_p)
    y = _post_call(agg2, hp2, dinv, b2.reshape(1, H), Wl, bl.reshape(1, 1))
    return y


# speedup vs baseline: 45.0069x; 1.1402x over previous
import functools

import jax
import jax.numpy as jnp
from jax import lax
from jax.experimental import pallas as pl
from jax.experimental.pallas import tpu as pltpu
from jax.experimental.pallas import tpu_sc as plsc

N = 10000
D = 128
H = 64
E = 320000

NC = 2
NS = 16
NW = NC * NS
C = 128
NCHUNK = E // C
NPAIR_TOT = NCHUNK // 2
PBASE = NPAIR_TOT // NW
PEXTRA = NPAIR_TOT - PBASE * NW
CHMAX = 2 * (PBASE + 1)
RPT = 640
NRP = NS * RPT

_mesh = plsc.VectorSubcoreMesh(
    core_axis_name="c", subcore_axis_name="s", num_cores=NC, num_subcores=NS
)

_sc_params = pltpu.CompilerParams(use_tc_tiling_on_sc=False)


def _load_idx_slab(ei3, which, cbase, wid, slab):
    pltpu.sync_copy(ei3.at[which, pl.ds(cbase, 2 * PBASE)],
                    slab.at[pl.ds(0, 2 * PBASE)])

    @pl.when(wid < PEXTRA)
    def _():
        pltpu.sync_copy(ei3.at[which, pl.ds(cbase + 2 * PBASE, 2)],
                        slab.at[pl.ds(2 * PBASE, 2)])


@functools.partial(
    pl.kernel,
    out_type=jax.ShapeDtypeStruct((NC, NRP), jnp.float32),
    mesh=_mesh,
    scratch_types=[
        pltpu.VMEM((CHMAX, C), jnp.int32),
        pltpu.VMEM((C,), jnp.float32),
        pltpu.VMEM((RPT,), jnp.float32),
        pltpu.VMEM_SHARED((NRP,), jnp.float32),
    ],
    compiler_params=_sc_params,
)
def _deg_kernel(ei3, out, idxs_v, ones_v, zb_v, acc):
    c = lax.axis_index("c")
    s = lax.axis_index("s")
    wid = c * NS + s
    npair = PBASE + jnp.where(wid < PEXTRA, 1, 0)
    cbase = 2 * (PBASE * wid + jnp.minimum(wid, PEXTRA))
    one = jnp.ones((16,), jnp.float32)
    zero = jnp.zeros((16,), jnp.float32)
    for q in range(C // 16):
        ones_v[pl.ds(q * 16, 16)] = one
    for q in range(RPT // 16):
        zb_v[pl.ds(q * 16, 16)] = zero
    _load_idx_slab(ei3, 1, cbase, wid, idxs_v)
    pltpu.sync_copy(zb_v, acc.at[pl.ds(s * RPT, RPT)])
    plsc.subcore_barrier()

    def body(j, carry):
        pltpu.sync_copy(ones_v, acc.at[idxs_v.at[j]], add=True)
        return carry

    lax.fori_loop(0, 2 * npair, body, 0)
    plsc.subcore_barrier()
    pltpu.sync_copy(acc.at[pl.ds(s * RPT, RPT)], out.at[c, pl.ds(s * RPT, RPT)])


@functools.partial(
    pl.kernel,
    out_type=jax.ShapeDtypeStruct((NC, NRP, H), jnp.float32),
    mesh=_mesh,
    scratch_types=[
        pltpu.VMEM((CHMAX, C), jnp.int32),
        pltpu.VMEM((CHMAX, C), jnp.int32),
        pltpu.VMEM((C, H), jnp.float32),
        pltpu.VMEM((C, H), jnp.float32),
        pltpu.VMEM((C, H), jnp.float32),
        pltpu.VMEM((C, H), jnp.float32),
        pltpu.VMEM((64, H), jnp.float32),
        pltpu.VMEM_SHARED((NRP, H), jnp.float32),
        pltpu.SemaphoreType.DMA,
        pltpu.SemaphoreType.DMA,
        pltpu.SemaphoreType.DMA,
        pltpu.SemaphoreType.DMA,
        pltpu.SemaphoreType.DMA,
        pltpu.SemaphoreType.DMA,
        pltpu.SemaphoreType.DMA,
        pltpu.SemaphoreType.DMA,
    ],
    compiler_params=_sc_params,
)
def _agg_kernel(hp, ei3, out, srcs_v, dsts_v, r0, r1, r2, r3, zb_v, acc,
                g0, g1, g2, g3, s0, s1, s2, s3):
    c = lax.axis_index("c")
    s = lax.axis_index("s")
    wid = c * NS + s
    npair = PBASE + jnp.where(wid < PEXTRA, 1, 0)
    nch = 2 * npair
    cbase = 2 * (PBASE * wid + jnp.minimum(wid, PEXTRA))
    zero = jnp.zeros((16,), jnp.float32)
    for r in range(64):
        for q in range(H // 16):
            zb_v[r, pl.ds(q * 16, 16)] = zero
    _load_idx_slab(ei3, 0, cbase, wid, srcs_v)
    _load_idx_slab(ei3, 1, cbase, wid, dsts_v)
    for k in range(RPT // 64):
        pltpu.sync_copy(zb_v, acc.at[pl.ds(s * RPT + k * 64, 64)])
    plsc.subcore_barrier()

    def gather(j, rb, gb):
        pltpu.async_copy(hp.at[srcs_v.at[j]], rb, gb)

    def wait_gather(rb, gb):
        pltpu.make_async_copy(hp.at[srcs_v.at[0]], rb, gb).wait()

    def scatter(j, rb, sb):
        pltpu.async_copy(rb, acc.at[dsts_v.at[j]], sb, add=True)

    def wait_scatter(rb, sb):
        pltpu.make_async_copy(rb, acc.at[dsts_v.at[0]], sb).wait()

    gather(0, r0, g0)
    gather(1, r1, g1)

    def phase(i, j, ra0, ra1, ga0, ga1, sa0, sa1, rb0, rb1, gb0, gb1, sb0, sb1):
        @pl.when(i > 0)
        def _():
            wait_scatter(rb0, sb0)
            wait_scatter(rb1, sb1)

        @pl.when(j + 2 < nch)
        def _():
            gather(j + 2, rb0, gb0)

        @pl.when(j + 3 < nch)
        def _():
            gather(j + 3, rb1, gb1)

        wait_gather(ra0, ga0)
        scatter(j, ra0, sa0)
        wait_gather(ra1, ga1)
        scatter(j + 1, ra1, sa1)

    def body(i, carry):
        j = 2 * i

        @pl.when(i % 2 == 0)
        def _():
            phase(i, j, r0, r1, g0, g1, s0, s1, r2, r3, g2, g3, s2, s3)

        @pl.when(i % 2 == 1)
        def _():
            phase(i, j, r2, r3, g2, g3, s2, s3, r0, r1, g0, g1, s0, s1)

        return carry

    lax.fori_loop(0, npair, body, 0)

    @pl.when((npair - 1) % 2 == 0)
    def _():
        wait_scatter(r0, s0)
        wait_scatter(r1, s1)

    @pl.when((npair - 1) % 2 == 1)
    def _():
        wait_scatter(r2, s2)
        wait_scatter(r3, s3)

    plsc.subcore_barrier()
    pltpu.sync_copy(acc.at[pl.ds(s * RPT, RPT)], out.at[c, pl.ds(s * RPT, RPT)])


def _pre_body(degp_ref, x_ref, w1_ref, hp_ref, dinv_ref):
    deg = degp_ref[0, :N, :] + degp_ref[1, :N, :] + 1.0
    dinv = lax.rsqrt(deg)
    h = jnp.dot(x_ref[...], w1_ref[...], preferred_element_type=jnp.float32)
    hp_ref[...] = h * dinv
    dinv_ref[...] = jnp.broadcast_to(dinv, (N, H))


_pre_call = pl.pallas_call(
    _pre_body,
    out_shape=(
        jax.ShapeDtypeStruct((N, H), jnp.float32),
        jax.ShapeDtypeStruct((N, H), jnp.float32),
    ),
)


def _mid_body(agg_ref, hp1_ref, dinv_ref, b1_ref, w2_ref, hp2_ref):
    a = agg_ref[0, :N, :] + agg_ref[1, :N, :]
    dinv = dinv_ref[...]
    z1 = jnp.maximum(dinv * (a + hp1_ref[...]) + b1_ref[...], 0.0)
    hp2_ref[...] = jnp.dot(z1, w2_ref[...], preferred_element_type=jnp.float32) * dinv


_mid_call = pl.pallas_call(
    _mid_body,
    out_shape=jax.ShapeDtypeStruct((N, H), jnp.float32),
)


def _post_body(agg_ref, hp2_ref, dinv_ref, b2_ref, wl_ref, bl_ref, y_ref):
    a = agg_ref[0, :N, :] + agg_ref[1, :N, :]
    dinv = dinv_ref[...]
    z2 = jnp.maximum(dinv * (a + hp2_ref[...]) + b2_ref[...], 0.0)
    y_ref[...] = jnp.dot(z2, wl_ref[...], preferred_element_type=jnp.float32) + bl_ref[...]


_post_call = pl.pallas_call(
    _post_body,
    out_shape=jax.ShapeDtypeStruct((N, 1), jnp.float32),
)


def kernel(x, edge_index, W1, b1, W2, b2, Wl, bl):
    ei3 = edge_index.reshape(2, NCHUNK, C)
    degp = _deg_kernel(ei3).reshape(NC, NRP, 1)
    hp1, dinv = _pre_call(degp, x, W1)
    agg1 = _agg_kernel(hp1, ei3)
    hp2 = _mid_call(agg1, hp1, dinv, b1.reshape(1, H), W2)
    agg2 = _agg_kernel(hp2, ei3)
    y = _post_call(agg2, hp2, dinv, b2.reshape(1, H), Wl, bl.reshape(1, 1))
    return y


# speedup vs baseline: 48.2163x; 1.0713x over previous
import functools

import jax
import jax.numpy as jnp
from jax import lax
from jax.experimental import pallas as pl
from jax.experimental.pallas import tpu as pltpu
from jax.experimental.pallas import tpu_sc as plsc

N = 10000
D = 128
H = 64
E = 320000

NC = 2
NS = 16
NW = NC * NS
C = 128
NCHUNK = E // C
NPAIR_TOT = NCHUNK // 2
PBASE = NPAIR_TOT // NW
PEXTRA = NPAIR_TOT - PBASE * NW
CHMAX = 2 * (PBASE + 1)
RPT = 640
NRP = NS * RPT

_mesh = plsc.VectorSubcoreMesh(
    core_axis_name="c", subcore_axis_name="s", num_cores=NC, num_subcores=NS
)

_sc_params = pltpu.CompilerParams(use_tc_tiling_on_sc=False)


def _load_idx_slab(ei3, which, cbase, wid, slab):
    pltpu.sync_copy(ei3.at[which, pl.ds(cbase, 2 * PBASE)],
                    slab.at[pl.ds(0, 2 * PBASE)])

    @pl.when(wid < PEXTRA)
    def _():
        pltpu.sync_copy(ei3.at[which, pl.ds(cbase + 2 * PBASE, 2)],
                        slab.at[pl.ds(2 * PBASE, 2)])


@functools.partial(
    pl.kernel,
    out_type=jax.ShapeDtypeStruct((NC, NRP), jnp.float32),
    mesh=_mesh,
    scratch_types=[
        pltpu.VMEM((CHMAX, C), jnp.int32),
        pltpu.VMEM((C,), jnp.float32),
        pltpu.VMEM((RPT,), jnp.float32),
        pltpu.VMEM_SHARED((NRP,), jnp.float32),
    ],
    compiler_params=_sc_params,
)
def _deg_kernel(ei3, out, idxs_v, ones_v, zb_v, acc):
    c = lax.axis_index("c")
    s = lax.axis_index("s")
    wid = c * NS + s
    npair = PBASE + jnp.where(wid < PEXTRA, 1, 0)
    cbase = 2 * (PBASE * wid + jnp.minimum(wid, PEXTRA))
    one = jnp.ones((16,), jnp.float32)
    zero = jnp.zeros((16,), jnp.float32)
    for q in range(C // 16):
        ones_v[pl.ds(q * 16, 16)] = one
    for q in range(RPT // 16):
        zb_v[pl.ds(q * 16, 16)] = zero
    _load_idx_slab(ei3, 1, cbase, wid, idxs_v)
    pltpu.sync_copy(zb_v, acc.at[pl.ds(s * RPT, RPT)])
    plsc.subcore_barrier()

    def body(j, carry):
        pltpu.sync_copy(ones_v, acc.at[idxs_v.at[j]], add=True)
        return carry

    lax.fori_loop(0, 2 * npair, body, 0)
    plsc.subcore_barrier()
    pltpu.sync_copy(acc.at[pl.ds(s * RPT, RPT)], out.at[c, pl.ds(s * RPT, RPT)])


@functools.partial(
    pl.kernel,
    out_type=jax.ShapeDtypeStruct((NC, NRP, H), jnp.float32),
    mesh=_mesh,
    scratch_types=[
        pltpu.VMEM((CHMAX, C), jnp.int32),
        pltpu.VMEM((CHMAX, C), jnp.int32),
        pltpu.VMEM((C, H), jnp.float32),
        pltpu.VMEM((C, H), jnp.float32),
        pltpu.VMEM((C, H), jnp.float32),
        pltpu.VMEM((C, H), jnp.float32),
        pltpu.VMEM((64, H), jnp.float32),
        pltpu.VMEM_SHARED((NRP, H), jnp.float32),
        pltpu.SemaphoreType.DMA,
        pltpu.SemaphoreType.DMA,
        pltpu.SemaphoreType.DMA,
        pltpu.SemaphoreType.DMA,
        pltpu.SemaphoreType.DMA,
        pltpu.SemaphoreType.DMA,
        pltpu.SemaphoreType.DMA,
        pltpu.SemaphoreType.DMA,
    ],
    compiler_params=_sc_params,
)
def _agg_kernel(hp, ei3, out, srcs_v, dsts_v, r0, r1, r2, r3, zb_v, acc,
                g0, g1, g2, g3, s0, s1, s2, s3):
    c = lax.axis_index("c")
    s = lax.axis_index("s")
    wid = c * NS + s
    npair = PBASE + jnp.where(wid < PEXTRA, 1, 0)
    nch = 2 * npair
    cbase = 2 * (PBASE * wid + jnp.minimum(wid, PEXTRA))
    zero = jnp.zeros((16,), jnp.float32)
    for r in range(64):
        for q in range(H // 16):
            zb_v[r, pl.ds(q * 16, 16)] = zero
    _load_idx_slab(ei3, 0, cbase, wid, srcs_v)
    _load_idx_slab(ei3, 1, cbase, wid, dsts_v)
    for k in range(RPT // 64):
        pltpu.sync_copy(zb_v, acc.at[pl.ds(s * RPT + k * 64, 64)])
    plsc.subcore_barrier()

    def gather(j, rb, gb):
        pltpu.async_copy(hp.at[srcs_v.at[j]], rb, gb)

    def wait_gather(rb, gb):
        pltpu.make_async_copy(hp.at[srcs_v.at[0]], rb, gb).wait()

    def scatter(j, rb, sb):
        pltpu.async_copy(rb, acc.at[dsts_v.at[j]], sb, add=True)

    def wait_scatter(rb, sb):
        pltpu.make_async_copy(rb, acc.at[dsts_v.at[0]], sb).wait()

    gather(0, r0, g0)
    gather(1, r1, g1)

    def phase(i, j, ra0, ra1, ga0, ga1, sa0, sa1, rb0, rb1, gb0, gb1, sb0, sb1):
        @pl.when(i > 0)
        def _():
            wait_scatter(rb0, sb0)
            wait_scatter(rb1, sb1)

        @pl.when(j + 2 < nch)
        def _():
            gather(j + 2, rb0, gb0)

        @pl.when(j + 3 < nch)
        def _():
            gather(j + 3, rb1, gb1)

        wait_gather(ra0, ga0)
        scatter(j, ra0, sa0)
        wait_gather(ra1, ga1)
        scatter(j + 1, ra1, sa1)

    def body(i, carry):
        j = 2 * i

        @pl.when(i % 2 == 0)
        def _():
            phase(i, j, r0, r1, g0, g1, s0, s1, r2, r3, g2, g3, s2, s3)

        @pl.when(i % 2 == 1)
        def _():
            phase(i, j, r2, r3, g2, g3, s2, s3, r0, r1, g0, g1, s0, s1)

        return carry

    lax.fori_loop(0, npair, body, 0)

    @pl.when((npair - 1) % 2 == 0)
    def _():
        wait_scatter(r0, s0)
        wait_scatter(r1, s1)

    @pl.when((npair - 1) % 2 == 1)
    def _():
        wait_scatter(r2, s2)
        wait_scatter(r3, s3)

    plsc.subcore_barrier()
    pltpu.sync_copy(acc.at[pl.ds(s * RPT, RPT)], out.at[c, pl.ds(s * RPT, RPT)])


def _pre_body(degp_ref, x_ref, w1_ref, hp_ref, dinv_ref):
    d = degp_ref[0] + degp_ref[1] + 1.0
    dv = lax.rsqrt(d)
    eye = jnp.eye(NRP // C, dtype=jnp.float32)
    t = lax.dot_general(dv, eye, (((0,), (0,)), ((), ())),
                        preferred_element_type=jnp.float32)
    h = jnp.dot(x_ref[...], w1_ref[...], preferred_element_type=jnp.float32)
    nfull = N // C
    for r in range(nfull):
        col = t[:, r:r + 1]
        blk = h[r * C:(r + 1) * C, :]
        hp_ref[pl.ds(r * C, C), :] = blk * col
        dinv_ref[pl.ds(r * C, C), :] = jnp.broadcast_to(col, (C, H))
    rem = N - nfull * C
    colt = t[:rem, nfull:nfull + 1]
    hp_ref[pl.ds(nfull * C, rem), :] = h[nfull * C:N, :] * colt
    dinv_ref[pl.ds(nfull * C, rem), :] = jnp.broadcast_to(colt, (rem, H))


_pre_call = pl.pallas_call(
    _pre_body,
    out_shape=(
        jax.ShapeDtypeStruct((N, H), jnp.float32),
        jax.ShapeDtypeStruct((N, H), jnp.float32),
    ),
)


def _mid_body(agg_ref, hp1_ref, dinv_ref, b1_ref, w2_ref, hp2_ref):
    a = agg_ref[0, :N, :] + agg_ref[1, :N, :]
    dinv = dinv_ref[...]
    z1 = jnp.maximum(dinv * (a + hp1_ref[...]) + b1_ref[...], 0.0)
    hp2_ref[...] = jnp.dot(z1, w2_ref[...], preferred_element_type=jnp.float32) * dinv


_mid_call = pl.pallas_call(
    _mid_body,
    out_shape=jax.ShapeDtypeStruct((N, H), jnp.float32),
)


def _post_body(agg_ref, hp2_ref, dinv_ref, b2_ref, wl_ref, bl_ref, y_ref):
    a = agg_ref[0, :N, :] + agg_ref[1, :N, :]
    dinv = dinv_ref[...]
    z2 = jnp.maximum(dinv * (a + hp2_ref[...]) + b2_ref[...], 0.0)
    y_ref[...] = jnp.dot(z2, wl_ref[...], preferred_element_type=jnp.float32) + bl_ref[...]


_post_call = pl.pallas_call(
    _post_body,
    out_shape=jax.ShapeDtypeStruct((N, 1), jnp.float32),
)


def kernel(x, edge_index, W1, b1, W2, b2, Wl, bl):
    ei3 = edge_index.reshape(2, NCHUNK, C)
    degp = _deg_kernel(ei3).reshape(NC, NRP // C, C)
    hp1, dinv = _pre_call(degp, x, W1)
    agg1 = _agg_kernel(hp1, ei3)
    hp2 = _mid_call(agg1, hp1, dinv, b1.reshape(1, H), W2)
    agg2 = _agg_kernel(hp2, ei3)
    y = _post_call(agg2, hp2, dinv, b2.reshape(1, H), Wl, bl.reshape(1, 1))
    return y
